# baseline (speedup 1.0000x reference)
import functools

import jax
import jax.numpy as jnp
from jax import lax
from jax.experimental import pallas as pl
from jax.experimental.pallas import tpu as pltpu
from jax.experimental.pallas import tpu_sc as plsc

N = 10000
E = 320000
D = 128
HID = 128
DH = D // 2

NC = 2
NS = 16
EPT = E // NS
K = 80
NSUB = 2
KK = K * NSUB
ROUNDS = EPT // KK
NMB = 4
NWB = 2
NRB = 4
RPT = 624
TAIL = N - NS * RPT


def _sc_agg_body(xs, ei, ew, zeros, out, m_v, w_v, rows_v, agg_sh, *sems):
    msem = sems[:NMB]
    gsem = sems[NMB:NMB + NRB]
    ssem = sems[NMB + NRB:]
    cid = lax.axis_index("c")
    sid = lax.axis_index("s")
    col = cid * DH

    pltpu.sync_copy(zeros.at[pl.ds(sid * RPT, RPT)],
                    agg_sh.at[pl.ds(sid * RPT, RPT)])

    @pl.when(sid == NS - 1)
    def _():
        pltpu.sync_copy(zeros.at[pl.ds(NS * RPT, TAIL)],
                        agg_sh.at[pl.ds(NS * RPT, TAIL)])

    plsc.subcore_barrier()

    def fire_meta(r, mb, wb):
        base = sid * EPT + r * KK
        for j in range(NSUB):
            pltpu.async_copy(ei.at[0, pl.ds(base + j * K, K)],
                             m_v.at[mb, 0, j], msem[mb])
            pltpu.async_copy(ei.at[1, pl.ds(base + j * K, K)],
                             m_v.at[mb, 1, j], msem[mb])
        pltpu.async_copy(ew.at[pl.ds(base, KK)], w_v.at[wb], msem[mb])

    def wait_meta(mb, wb):
        for j in range(NSUB):
            pltpu.make_async_copy(ei.at[0, pl.ds(0, K)],
                                  m_v.at[mb, 0, j], msem[mb]).wait()
            pltpu.make_async_copy(ei.at[1, pl.ds(0, K)],
                                  m_v.at[mb, 1, j], msem[mb]).wait()
        pltpu.make_async_copy(ew.at[pl.ds(0, KK)], w_v.at[wb],
                              msem[mb]).wait()
        off = cid * N
        for j in range(NSUB):
            for g in range(K // 16):
                sl = (mb, 0, j, pl.ds(g * 16, 16))
                m_v[sl] = m_v[sl] + off

    def fire_gathers(mb, rb):
        for j in range(NSUB):
            pltpu.async_copy(xs.at[m_v.at[mb, 0, j]],
                             rows_v.at[rb, pl.ds(j * K, K)], gsem[rb])

    def wait_gathers(mb, rb):
        for j in range(NSUB):
            pltpu.make_async_copy(xs.at[m_v.at[mb, 0, j]],
                                  rows_v.at[rb, pl.ds(j * K, K)],
                                  gsem[rb]).wait()

    def fire_scatters(mb, rb):
        for j in range(NSUB):
            pltpu.async_copy(rows_v.at[rb, pl.ds(j * K, K)],
                             agg_sh.at[m_v.at[mb, 1, j]], ssem[rb], add=True)

    def wait_scatters(mb, rb):
        for j in range(NSUB):
            pltpu.make_async_copy(rows_v.at[rb, pl.ds(j * K, K)],
                                  agg_sh.at[m_v.at[mb, 1, j]],
                                  ssem[rb]).wait()

    def scale(wb, rb):
        def body(g, carry):
            w16 = w_v[wb, pl.ds(g * 16, 16)]
            for l in range(16):
                e = g * 16 + l
                wl = lax.gather(
                    w16, jnp.full((16, 1), l, jnp.int32),
                    lax.GatherDimensionNumbers(
                        offset_dims=(), collapsed_slice_dims=(0,),
                        start_index_map=(0,)),
                    slice_sizes=(1,),
                    mode=lax.GatherScatterMode.PROMISE_IN_BOUNDS)
                for j in range(DH // 16):
                    sl = (rb, e, pl.ds(j * 16, 16))
                    rows_v[sl] = rows_v[sl] * wl
            return carry

        lax.fori_loop(0, KK // 16, body, 0)

    def do_round(r, pr, drain=True, nxt=True, nxt2=True):
        b, mb, wb = pr % NRB, pr % NMB, pr % NWB
        if nxt:
            wait_meta((pr + 1) % NMB, (pr + 1) % NWB)
        if drain:
            wait_scatters((pr - 2) % NMB, (pr - 2) % NRB)
        if nxt:
            fire_gathers((pr + 1) % NMB, (pr + 1) % NRB)
        wait_gathers(mb, b)
        scale(wb, b)
        if nxt2:
            fire_meta(r + 2, (pr + 2) % NMB, (pr + 2) % NWB)
        fire_scatters(mb, b)

    fire_meta(0, 0, 0)
    fire_meta(1, 1, 1)
    wait_meta(0, 0)
    fire_gathers(0, 0)
    do_round(0, 0, drain=False)
    do_round(1, 1, drain=False)

    def chunk4(t, carry):
        for i in range(4):
            do_round(2 + 4 * t + i, 2 + i)
        return carry

    lax.fori_loop(0, (ROUNDS - 5) // 4, chunk4, 0)
    do_round(122, 122)
    do_round(123, 123, nxt2=False)
    do_round(124, 124, nxt=False, nxt2=False)
    wait_scatters(123 % NMB, 123 % NRB)
    wait_scatters(124 % NMB, 124 % NRB)

    plsc.subcore_barrier()
    pltpu.sync_copy(agg_sh.at[pl.ds(sid * RPT, RPT)],
                    out.at[pl.ds(sid * RPT, RPT), pl.ds(col, DH)])

    @pl.when(sid == NS - 1)
    def _():
        pltpu.sync_copy(agg_sh.at[pl.ds(NS * RPT, TAIL)],
                        out.at[pl.ds(NS * RPT, TAIL), pl.ds(col, DH)])


_sc_agg = functools.partial(
    pl.kernel,
    out_type=jax.ShapeDtypeStruct((N, D), jnp.float32),
    mesh=plsc.VectorSubcoreMesh(
        core_axis_name="c", subcore_axis_name="s",
        num_cores=NC, num_subcores=NS),
    scratch_types=[
        pltpu.VMEM((NMB, 2, NSUB, K), jnp.int32),
        pltpu.VMEM((NWB, KK), jnp.float32),
        pltpu.VMEM((NRB, KK, DH), jnp.float32),
        pltpu.VMEM_SHARED((N, DH), jnp.float32),
    ] + [pltpu.SemaphoreType.DMA] * (NMB + 2 * NRB),
    compiler_params=pltpu.CompilerParams(use_tc_tiling_on_sc=False),
)(_sc_agg_body)


_BM = 1000


def _dense_body(p_ref, x_ref, wrel_ref, b_ref, wroot_ref, o_ref, os_ref):
    h = lax.dot_general(p_ref[...], wrel_ref[...], (((1,), (1,)), ((), ())),
                        preferred_element_type=jnp.float32)
    h = h + lax.dot_general(x_ref[...], wroot_ref[...],
                            (((1,), (1,)), ((), ())),
                            preferred_element_type=jnp.float32)
    h = jnp.maximum(h + b_ref[...], 0.0)
    o_ref[...] = h
    os_ref[0] = h[:, :DH]
    os_ref[1] = h[:, DH:]


def _dense(p, x, w_rel, b_rel, w_root):
    return pl.pallas_call(
        _dense_body,
        grid=(N // _BM,),
        in_specs=[
            pl.BlockSpec((_BM, D), lambda i: (i, 0)),
            pl.BlockSpec((_BM, D), lambda i: (i, 0)),
            pl.BlockSpec((HID, D), lambda i: (0, 0)),
            pl.BlockSpec((1, HID), lambda i: (0, 0)),
            pl.BlockSpec((HID, HID), lambda i: (0, 0)),
        ],
        out_specs=[
            pl.BlockSpec((_BM, HID), lambda i: (i, 0)),
            pl.BlockSpec((2, _BM, DH), lambda i: (0, i, 0)),
        ],
        out_shape=[
            jax.ShapeDtypeStruct((N, HID), jnp.float32),
            jax.ShapeDtypeStruct((2, N, DH), jnp.float32),
        ],
    )(p, x, w_rel, b_rel, w_root)


def _split_cols(a):
    return jnp.concatenate([a[:, :DH], a[:, DH:]], axis=0)


def kernel(x, edge_index, edge_weight, W1_rel, b1_rel, W1_root,
           W2_rel, b2_rel, W2_root):
    ei = edge_index.astype(jnp.int32)
    ew = edge_weight.astype(jnp.float32)
    zeros = jnp.zeros((N, DH), jnp.float32)

    p1 = _sc_agg(_split_cols(x), ei, ew, zeros)
    h1, h1s = _dense(p1, x, W1_rel, b1_rel.reshape(1, HID), W1_root)
    p2 = _sc_agg(h1s.reshape(NC * N, DH), ei, ew, zeros)
    h2, _ = _dense(p2, h1, W2_rel, b2_rel.reshape(1, HID), W2_root)
    return h2


# speedup vs baseline: 1.0945x; 1.0945x over previous
import functools

import jax
import jax.numpy as jnp
from jax import lax
from jax.experimental import pallas as pl
from jax.experimental.pallas import tpu as pltpu
from jax.experimental.pallas import tpu_sc as plsc

N = 10000
E = 320000
D = 128
HID = 128
DH = D // 2

NC = 2
NS = 16
EPT = E // NS
K = 80
NSUB = 2
KK = K * NSUB
ROUNDS = EPT // KK
NMB = 4
NWB = 2
NRB = 4
RPT = 624
TAIL = N - NS * RPT


def _sc_agg_body(xs, ei, w, zeros, out, m_v, w_v, rows_v, agg_sh, *sems):
    msem = sems[:NMB]
    gsem = sems[NMB:NMB + NRB]
    ssem = sems[NMB + NRB:]
    cid = lax.axis_index("c")
    sid = lax.axis_index("s")
    col = cid * DH

    pltpu.sync_copy(zeros.at[pl.ds(sid * RPT, RPT)],
                    agg_sh.at[pl.ds(sid * RPT, RPT)])

    @pl.when(sid == NS - 1)
    def _():
        pltpu.sync_copy(zeros.at[pl.ds(NS * RPT, TAIL)],
                        agg_sh.at[pl.ds(NS * RPT, TAIL)])

    plsc.subcore_barrier()

    def fire_meta(r, mb, wb):
        base = sid * EPT + r * KK
        for j in range(NSUB):
            pltpu.async_copy(ei.at[0, pl.ds(base + j * K, K)],
                             m_v.at[mb, 0, j], msem[mb])
            pltpu.async_copy(ei.at[1, pl.ds(base + j * K, K)],
                             m_v.at[mb, 1, j], msem[mb])
        pltpu.async_copy(w.at[sid, r], w_v.at[wb], msem[mb])

    def wait_meta(mb, wb):
        for j in range(NSUB):
            pltpu.make_async_copy(ei.at[0, pl.ds(0, K)],
                                  m_v.at[mb, 0, j], msem[mb]).wait()
            pltpu.make_async_copy(ei.at[1, pl.ds(0, K)],
                                  m_v.at[mb, 1, j], msem[mb]).wait()
        pltpu.make_async_copy(w.at[sid, 0], w_v.at[wb], msem[mb]).wait()
        off = cid * N
        for j in range(NSUB):
            for g in range(K // 16):
                sl = (mb, 0, j, pl.ds(g * 16, 16))
                m_v[sl] = m_v[sl] + off

    def fire_gathers(mb, rb):
        for j in range(NSUB):
            pltpu.async_copy(xs.at[m_v.at[mb, 0, j]],
                             rows_v.at[rb, pl.ds(j * K, K)], gsem[rb])

    def wait_gathers(mb, rb):
        for j in range(NSUB):
            pltpu.make_async_copy(xs.at[m_v.at[mb, 0, j]],
                                  rows_v.at[rb, pl.ds(j * K, K)],
                                  gsem[rb]).wait()

    def fire_scatters(mb, rb):
        for j in range(NSUB):
            pltpu.async_copy(rows_v.at[rb, pl.ds(j * K, K)],
                             agg_sh.at[m_v.at[mb, 1, j]], ssem[rb], add=True)

    def wait_scatters(mb, rb):
        for j in range(NSUB):
            pltpu.make_async_copy(rows_v.at[rb, pl.ds(j * K, K)],
                                  agg_sh.at[m_v.at[mb, 1, j]],
                                  ssem[rb]).wait()

    def scale(wb, rb):
        def body(e, carry):
            we = w_v[wb, pl.ds(e * 16, 16)]
            for j in range(DH // 16):
                sl = (rb, e, pl.ds(j * 16, 16))
                rows_v[sl] = rows_v[sl] * we
            return carry

        lax.fori_loop(0, KK, body, 0, unroll=4)

    def do_round(r, pr, drain=True, nxt=True, nxt2=True):
        b, mb, wb = pr % NRB, pr % NMB, pr % NWB
        if nxt:
            wait_meta((pr + 1) % NMB, (pr + 1) % NWB)
        if drain:
            wait_scatters((pr - 2) % NMB, (pr - 2) % NRB)
        if nxt:
            fire_gathers((pr + 1) % NMB, (pr + 1) % NRB)
        wait_gathers(mb, b)
        scale(wb, b)
        if nxt2:
            fire_meta(r + 2, (pr + 2) % NMB, (pr + 2) % NWB)
        fire_scatters(mb, b)

    fire_meta(0, 0, 0)
    fire_meta(1, 1, 1)
    wait_meta(0, 0)
    fire_gathers(0, 0)
    do_round(0, 0, drain=False)
    do_round(1, 1, drain=False)

    def chunk4(t, carry):
        for i in range(4):
            do_round(2 + 4 * t + i, 2 + i)
        return carry

    lax.fori_loop(0, (ROUNDS - 5) // 4, chunk4, 0)
    do_round(122, 122)
    do_round(123, 123, nxt2=False)
    do_round(124, 124, nxt=False, nxt2=False)
    wait_scatters(123 % NMB, 123 % NRB)
    wait_scatters(124 % NMB, 124 % NRB)

    plsc.subcore_barrier()
    pltpu.sync_copy(agg_sh.at[pl.ds(sid * RPT, RPT)],
                    out.at[pl.ds(sid * RPT, RPT), pl.ds(col, DH)])

    @pl.when(sid == NS - 1)
    def _():
        pltpu.sync_copy(agg_sh.at[pl.ds(NS * RPT, TAIL)],
                        out.at[pl.ds(NS * RPT, TAIL), pl.ds(col, DH)])


_sc_agg = functools.partial(
    pl.kernel,
    out_type=jax.ShapeDtypeStruct((N, D), jnp.float32),
    mesh=plsc.VectorSubcoreMesh(
        core_axis_name="c", subcore_axis_name="s",
        num_cores=NC, num_subcores=NS),
    scratch_types=[
        pltpu.VMEM((NMB, 2, NSUB, K), jnp.int32),
        pltpu.VMEM((NWB, KK * 16), jnp.float32),
        pltpu.VMEM((NRB, KK, DH), jnp.float32),
        pltpu.VMEM_SHARED((N, DH), jnp.float32),
    ] + [pltpu.SemaphoreType.DMA] * (NMB + 2 * NRB),
    compiler_params=pltpu.CompilerParams(use_tc_tiling_on_sc=False),
)(_sc_agg_body)


_BM = 1000


def _dense_body(p_ref, x_ref, wrel_ref, b_ref, wroot_ref, o_ref, os_ref):
    h = lax.dot_general(p_ref[...], wrel_ref[...], (((1,), (1,)), ((), ())),
                        preferred_element_type=jnp.float32)
    h = h + lax.dot_general(x_ref[...], wroot_ref[...],
                            (((1,), (1,)), ((), ())),
                            preferred_element_type=jnp.float32)
    h = jnp.maximum(h + b_ref[...], 0.0)
    o_ref[...] = h
    os_ref[0] = h[:, :DH]
    os_ref[1] = h[:, DH:]


def _dense(p, x, w_rel, b_rel, w_root):
    return pl.pallas_call(
        _dense_body,
        grid=(N // _BM,),
        in_specs=[
            pl.BlockSpec((_BM, D), lambda i: (i, 0)),
            pl.BlockSpec((_BM, D), lambda i: (i, 0)),
            pl.BlockSpec((HID, D), lambda i: (0, 0)),
            pl.BlockSpec((1, HID), lambda i: (0, 0)),
            pl.BlockSpec((HID, HID), lambda i: (0, 0)),
        ],
        out_specs=[
            pl.BlockSpec((_BM, HID), lambda i: (i, 0)),
            pl.BlockSpec((2, _BM, DH), lambda i: (0, i, 0)),
        ],
        out_shape=[
            jax.ShapeDtypeStruct((N, HID), jnp.float32),
            jax.ShapeDtypeStruct((2, N, DH), jnp.float32),
        ],
    )(p, x, w_rel, b_rel, w_root)


def _split_cols(a):
    return jnp.concatenate([a[:, :DH], a[:, DH:]], axis=0)


def kernel(x, edge_index, edge_weight, W1_rel, b1_rel, W1_root,
           W2_rel, b2_rel, W2_root):
    ei = edge_index.astype(jnp.int32)
    w = jnp.broadcast_to(
        edge_weight.astype(jnp.float32).reshape(NS, ROUNDS, KK)[..., None],
        (NS, ROUNDS, KK, 16)).reshape(NS, ROUNDS, KK * 16)
    zeros = jnp.zeros((N, DH), jnp.float32)

    p1 = _sc_agg(_split_cols(x), ei, w, zeros)
    h1, h1s = _dense(p1, x, W1_rel, b1_rel.reshape(1, HID), W1_root)
    p2 = _sc_agg(h1s.reshape(NC * N, DH), ei, w, zeros)
    h2, _ = _dense(p2, h1, W2_rel, b2_rel.reshape(1, HID), W2_root)
    return h2
